# Initial kernel scaffold; baseline (speedup 1.0000x reference)
import jax
import jax.numpy as jnp
from jax.experimental import pallas as pl


def kernel(x, edge_index, batch, W1, b1, W2, b2, W3, b3, Wl, bl):
    raise NotImplementedError("write your pallas kernel here")


# speedup vs baseline: 11.4081x; 11.4081x over previous
import functools

import jax
import jax.numpy as jnp
from jax import lax
from jax.experimental import pallas as pl
from jax.experimental.pallas import tpu as pltpu
from jax.experimental.pallas import tpu_sc as plsc

N_NODES = 10000
N_EDGES = 320000
D = 128
G = 64

NC = 2
NS = 16
NW = NC * NS

E_PER_W = N_EDGES // NW
CHUNK = 80
NCHUNK = E_PER_W // CHUNK
N_PAD = 10240
ROWS_PER_TILE = N_PAD // NS
ZROWS = 128

DEG_PAD = N_PAD

_mesh = plsc.VectorSubcoreMesh(core_axis_name="c", subcore_axis_name="s")

R = 1000
NBLK = N_NODES // R


@functools.partial(
    pl.kernel,
    out_type=jax.ShapeDtypeStruct((NC, DEG_PAD), jnp.float32),
    mesh=_mesh,
    scratch_types=[
        pltpu.VMEM((CHUNK,), jnp.int32),
        pltpu.VMEM((CHUNK,), jnp.float32),
        pltpu.VMEM((640,), jnp.float32),
        pltpu.VMEM_SHARED((DEG_PAD,), jnp.float32),
        pltpu.SemaphoreType.DMA,
    ],
)
def _deg_sc(col_hbm, out_hbm, cidx, ones_v, zeros_v, acc, sem):
    c = lax.axis_index("c")
    s = lax.axis_index("s")
    w = s * NC + c

    def _fill(i, _):
        zeros_v[pl.ds(i * 16, 16)] = jnp.zeros((16,), jnp.float32)
        return 0

    lax.fori_loop(0, 40, _fill, 0)

    def _fill1(i, _):
        ones_v[pl.ds(i * 16, 16)] = jnp.ones((16,), jnp.float32)
        return 0

    lax.fori_loop(0, CHUNK // 16, _fill1, 0)

    pltpu.sync_copy(zeros_v, acc.at[pl.ds(s * 640, 640)])
    plsc.subcore_barrier()

    def _body(j, _):
        base = w * E_PER_W + j * CHUNK
        pltpu.sync_copy(col_hbm.at[pl.ds(base, CHUNK)], cidx)
        pltpu.sync_copy(ones_v, acc.at[cidx], add=True)
        return 0

    lax.fori_loop(0, NCHUNK, _body, 0)
    plsc.subcore_barrier()
    pltpu.sync_copy(acc.at[pl.ds(s * 640, 640)], out_hbm.at[c, pl.ds(s * 640, 640)])


@functools.partial(
    pl.kernel,
    out_type=jax.ShapeDtypeStruct((NC, N_PAD, D), jnp.float32),
    mesh=_mesh,
    scratch_types=[
        pltpu.VMEM((CHUNK,), jnp.int32),
        pltpu.VMEM((CHUNK,), jnp.int32),
        pltpu.VMEM((CHUNK, D), jnp.float32),
        pltpu.VMEM((ZROWS, D), jnp.float32),
        pltpu.VMEM_SHARED((N_PAD, D), jnp.float32),
        pltpu.SemaphoreType.DMA,
    ],
)
def _spmm_sc(row_hbm, col_hbm, t_hbm, out_hbm, ridx, cidx, rows, zbuf, acc, sem):
    c = lax.axis_index("c")
    s = lax.axis_index("s")
    w = s * NC + c

    def _fill(i, _):
        for k in range(D // 16):
            zbuf[i, pl.ds(k * 16, 16)] = jnp.zeros((16,), jnp.float32)
        return 0

    lax.fori_loop(0, ZROWS, _fill, 0)
    for j in range(ROWS_PER_TILE // ZROWS):
        pltpu.sync_copy(zbuf, acc.at[pl.ds(s * ROWS_PER_TILE + j * ZROWS, ZROWS), :])
    plsc.subcore_barrier()

    def _body(j, _):
        base = w * E_PER_W + j * CHUNK
        pltpu.sync_copy(row_hbm.at[pl.ds(base, CHUNK)], ridx)
        pltpu.sync_copy(col_hbm.at[pl.ds(base, CHUNK)], cidx)
        pltpu.async_copy(t_hbm.at[ridx], rows, sem).wait()
        pltpu.sync_copy(rows, acc.at[cidx], add=True)
        return 0

    lax.fori_loop(0, NCHUNK, _body, 0)
    plsc.subcore_barrier()
    base = s * ROWS_PER_TILE
    pltpu.sync_copy(acc.at[pl.ds(base, ROWS_PER_TILE), :],
                    out_hbm.at[c, pl.ds(base, ROWS_PER_TILE), :])


def _dinv_of(deg_ref):
    d = deg_ref[0] + deg_ref[1] + 1.0
    return 1.0 / jnp.sqrt(d)


def _tc_first_body(x_ref, w_ref, deg_ref, out_ref):
    dinv = _dinv_of(deg_ref)
    t = jnp.dot(x_ref[...], w_ref[...], preferred_element_type=jnp.float32)
    out_ref[...] = t * dinv


def _tc_mid_body(a_ref, tp_ref, deg_ref, b_ref, w_ref, out_ref):
    dinv = _dinv_of(deg_ref)
    a = a_ref[0] + a_ref[1] + tp_ref[...]
    h = jnp.maximum(a * dinv + b_ref[...], 0.0)
    out_ref[...] = jnp.dot(h, w_ref[...], preferred_element_type=jnp.float32) * dinv


def _tc_pool_body(a_ref, tp_ref, deg_ref, b_ref, wl_ref, bl_ref, batch_ref,
                  out_ref, cnt_ref):
    i = pl.program_id(0)
    dinv = _dinv_of(deg_ref)
    h = (a_ref[0] + a_ref[1] + tp_ref[...]) * dinv + b_ref[...]
    y = jnp.dot(h, wl_ref[...], preferred_element_type=jnp.float32)
    gids = lax.broadcasted_iota(jnp.int32, (R, G), 1)
    p = (batch_ref[...] == gids).astype(jnp.float32)
    dn = (((0,), (0,)), ((), ()))
    ysum = lax.dot_general(p, y, dn, preferred_element_type=jnp.float32)
    csum = lax.dot_general(p, jnp.ones((R, 1), jnp.float32), dn,
                           preferred_element_type=jnp.float32)

    @pl.when(i == 0)
    def _():
        out_ref[...] = ysum
        cnt_ref[...] = csum

    @pl.when(i > 0)
    def _():
        out_ref[...] += ysum
        cnt_ref[...] += csum

    @pl.when(i == NBLK - 1)
    def _():
        out_ref[...] = out_ref[...] / jnp.maximum(cnt_ref[...], 1.0) + bl_ref[...]


_deg_spec = pl.BlockSpec((2, R, 1), lambda i: (0, i, 0))
_w_spec = pl.BlockSpec((D, D), lambda i: (0, 0))
_b_spec = pl.BlockSpec((1, D), lambda i: (0, 0))
_row_spec = pl.BlockSpec((R, D), lambda i: (i, 0))
_ap_spec = pl.BlockSpec((2, R, D), lambda i: (0, i, 0))

_tc_first = pl.pallas_call(
    _tc_first_body,
    grid=(NBLK,),
    in_specs=[_row_spec, _w_spec, _deg_spec],
    out_specs=_row_spec,
    out_shape=jax.ShapeDtypeStruct((N_NODES, D), jnp.float32),
)

_tc_mid = pl.pallas_call(
    _tc_mid_body,
    grid=(NBLK,),
    in_specs=[_ap_spec, _row_spec, _deg_spec, _b_spec, _w_spec],
    out_specs=_row_spec,
    out_shape=jax.ShapeDtypeStruct((N_NODES, D), jnp.float32),
)

_tc_pool = pl.pallas_call(
    _tc_pool_body,
    grid=(NBLK,),
    in_specs=[_ap_spec, _row_spec, _deg_spec, _b_spec,
              pl.BlockSpec((D, 1), lambda i: (0, 0)),
              pl.BlockSpec((1, 1), lambda i: (0, 0)),
              pl.BlockSpec((R, 1), lambda i: (i, 0))],
    out_specs=pl.BlockSpec((G, 1), lambda i: (0, 0)),
    out_shape=jax.ShapeDtypeStruct((G, 1), jnp.float32),
    scratch_shapes=[pltpu.VMEM((G, 1), jnp.float32)],
)


def kernel(x, edge_index, batch, W1, b1, W2, b2, W3, b3, Wl, bl):
    row = edge_index[0]
    col = edge_index[1]
    deg_p = _deg_sc(col)
    deg3 = deg_p[:, :N_NODES].reshape(2, N_NODES, 1)
    t1 = _tc_first(x, W1, deg3)
    a1 = _spmm_sc(row, col, t1)
    t2 = _tc_mid(a1, t1, deg3, b1.reshape(1, D), W2)
    a2 = _spmm_sc(row, col, t2)
    t3 = _tc_mid(a2, t2, deg3, b2.reshape(1, D), W3)
    a3 = _spmm_sc(row, col, t3)
    out = _tc_pool(a3, t3, deg3, b3.reshape(1, D), Wl,
                   bl.reshape(1, 1), batch.reshape(N_NODES, 1))
    return out
